# baseline (speedup 1.0000x reference)
import functools

import numpy as np
import jax
import jax.numpy as jnp
from jax.experimental import pallas as pl
from jax.experimental.pallas import tpu as pltpu

_E = 64
_TOPK = 8
_MAX_EXPERT = 0
_TARGET_VALUE = 64


def _rotl(v, d):
    return ((v << np.uint32(d)) | (v >> np.uint32(32 - d))).astype(np.uint32)


def _threefry2x32(k0, k1, x0, x1):
    ks0, ks1 = np.uint32(k0), np.uint32(k1)
    ks2 = np.uint32(ks0 ^ ks1 ^ np.uint32(0x1BD11BDA))
    rot0 = (13, 15, 26, 6)
    rot1 = (17, 29, 16, 24)
    x0 = (x0 + ks0).astype(np.uint32)
    x1 = (x1 + ks1).astype(np.uint32)

    def rounds(x0, x1, rots):
        for r in rots:
            x0 = (x0 + x1).astype(np.uint32)
            x1 = _rotl(x1, r)
            x1 = (x1 ^ x0).astype(np.uint32)
        return x0, x1

    x0, x1 = rounds(x0, x1, rot0)
    x0 = (x0 + ks1).astype(np.uint32); x1 = (x1 + ks2 + np.uint32(1)).astype(np.uint32)
    x0, x1 = rounds(x0, x1, rot1)
    x0 = (x0 + ks2).astype(np.uint32); x1 = (x1 + ks0 + np.uint32(2)).astype(np.uint32)
    x0, x1 = rounds(x0, x1, rot0)
    x0 = (x0 + ks0).astype(np.uint32); x1 = (x1 + ks1 + np.uint32(3)).astype(np.uint32)
    x0, x1 = rounds(x0, x1, rot1)
    x0 = (x0 + ks1).astype(np.uint32); x1 = (x1 + ks2 + np.uint32(4)).astype(np.uint32)
    x0, x1 = rounds(x0, x1, rot0)
    x0 = (x0 + ks2).astype(np.uint32); x1 = (x1 + ks0 + np.uint32(5)).astype(np.uint32)
    return x0, x1


@functools.lru_cache(maxsize=None)
def _replacement_table_t(n_tokens: int, k: int):
    b1, b2 = _threefry2x32(np.uint32(0), np.uint32(1),
                           np.zeros(2, np.uint32), np.arange(2, dtype=np.uint32))
    sk0, sk1 = b1[1], b2[1]
    size = n_tokens * k
    z = np.zeros(size, dtype=np.uint32)
    lo = np.arange(size, dtype=np.uint32)
    c1, c2 = _threefry2x32(sk0, sk1, z, lo)
    bit = ((c1 ^ c2) & np.uint32(1)).astype(np.int32)
    tab = np.where(bit == 0, _MAX_EXPERT, _TARGET_VALUE).astype(np.int32)
    return np.ascontiguousarray(tab.reshape(n_tokens, k).T)


def _gate_block(x_ref, w_ref, rep_ref, idx_ref, wgt_ref):
    f32, bf16, i32 = jnp.float32, jnp.bfloat16, jnp.int32
    x = x_ref[...]
    w = w_ref[...]
    logits = jax.lax.dot_general(
        w, x, (((1,), (1,)), ((), ())),
        preferred_element_type=f32).astype(bf16)
    m = jnp.max(logits, axis=0, keepdims=True)
    l32 = logits.astype(f32)
    m32 = jnp.broadcast_to(m, logits.shape).astype(f32)
    e32 = jnp.exp(l32 - m32)
    den = jnp.sum(e32, axis=0, keepdims=True).astype(bf16)
    s32 = e32 / jnp.broadcast_to(den, e32.shape).astype(f32)

    bits = jax.lax.bitcast_convert_type(s32, i32)
    bits = (bits & jnp.int32(0x7FFFFFFF)) ^ (bits >> 31)
    iota = jax.lax.broadcasted_iota(i32, s32.shape, 0)
    key = (bits | jnp.int32(0xFFFF)) ^ iota

    neg = jnp.int32(-2147483648)
    keys8 = []
    for _ in range(_TOPK):
        kmax = jnp.max(key, axis=0, keepdims=True)
        keys8.append(kmax)
        key = jnp.where(key == jnp.broadcast_to(kmax, key.shape), neg, key)
    k8 = jnp.concatenate(keys8, axis=0)
    ti = (k8 ^ jnp.int32(0xFFFF)) & jnp.int32(0xFFFF)
    wbits = ((k8 & jnp.int32(0x7FFFFFFF)) ^ (k8 >> 31)) & jnp.int32(-65536)
    tw32 = jax.lax.bitcast_convert_type(wbits, f32)
    denom = jnp.sum(tw32, axis=0, keepdims=True).astype(bf16)
    tv = (tw32 / jnp.broadcast_to(denom, tw32.shape).astype(f32)).astype(bf16)
    ti = jnp.where(ti == _MAX_EXPERT, rep_ref[...], ti)
    idx_ref[...] = ti.T
    wgt_ref[...] = tv.T


def kernel(hidden_states, weight):
    bsz, seq, h = hidden_states.shape
    n = bsz * seq
    x = hidden_states.reshape(n, h)
    rep = jnp.asarray(_replacement_table_t(n, _TOPK))

    bm = 1024
    grid = (n // bm,)
    ti, tv = pl.pallas_call(
        _gate_block,
        grid=grid,
        in_specs=[
            pl.BlockSpec((bm, h), lambda i: (i, 0)),
            pl.BlockSpec((_E, h), lambda i: (0, 0)),
            pl.BlockSpec((_TOPK, bm), lambda i: (0, i)),
        ],
        out_specs=[
            pl.BlockSpec((bm, _TOPK), lambda i: (i, 0)),
            pl.BlockSpec((bm, _TOPK), lambda i: (i, 0)),
        ],
        out_shape=[
            jax.ShapeDtypeStruct((n, _TOPK), jnp.int32),
            jax.ShapeDtypeStruct((n, _TOPK), hidden_states.dtype),
        ],
        compiler_params=pltpu.CompilerParams(
            dimension_semantics=("parallel",),
        ),
    )(x, weight, rep)
    return ti, tv


# speedup vs baseline: 1.2153x; 1.2153x over previous
import functools

import numpy as np
import jax
import jax.numpy as jnp
from jax.experimental import pallas as pl
from jax.experimental.pallas import tpu as pltpu

_E = 64
_TOPK = 8
_MAX_EXPERT = 0
_TARGET_VALUE = 64


def _rotl(v, d):
    return ((v << np.uint32(d)) | (v >> np.uint32(32 - d))).astype(np.uint32)


def _threefry2x32(k0, k1, x0, x1):
    ks0, ks1 = np.uint32(k0), np.uint32(k1)
    ks2 = np.uint32(ks0 ^ ks1 ^ np.uint32(0x1BD11BDA))
    rot0 = (13, 15, 26, 6)
    rot1 = (17, 29, 16, 24)
    x0 = (x0 + ks0).astype(np.uint32)
    x1 = (x1 + ks1).astype(np.uint32)

    def rounds(x0, x1, rots):
        for r in rots:
            x0 = (x0 + x1).astype(np.uint32)
            x1 = _rotl(x1, r)
            x1 = (x1 ^ x0).astype(np.uint32)
        return x0, x1

    x0, x1 = rounds(x0, x1, rot0)
    x0 = (x0 + ks1).astype(np.uint32); x1 = (x1 + ks2 + np.uint32(1)).astype(np.uint32)
    x0, x1 = rounds(x0, x1, rot1)
    x0 = (x0 + ks2).astype(np.uint32); x1 = (x1 + ks0 + np.uint32(2)).astype(np.uint32)
    x0, x1 = rounds(x0, x1, rot0)
    x0 = (x0 + ks0).astype(np.uint32); x1 = (x1 + ks1 + np.uint32(3)).astype(np.uint32)
    x0, x1 = rounds(x0, x1, rot1)
    x0 = (x0 + ks1).astype(np.uint32); x1 = (x1 + ks2 + np.uint32(4)).astype(np.uint32)
    x0, x1 = rounds(x0, x1, rot0)
    x0 = (x0 + ks2).astype(np.uint32); x1 = (x1 + ks0 + np.uint32(5)).astype(np.uint32)
    return x0, x1


@functools.lru_cache(maxsize=None)
def _replacement_table_t(n_tokens: int, k: int):
    b1, b2 = _threefry2x32(np.uint32(0), np.uint32(1),
                           np.zeros(2, np.uint32), np.arange(2, dtype=np.uint32))
    sk0, sk1 = b1[1], b2[1]
    size = n_tokens * k
    z = np.zeros(size, dtype=np.uint32)
    lo = np.arange(size, dtype=np.uint32)
    c1, c2 = _threefry2x32(sk0, sk1, z, lo)
    bit = ((c1 ^ c2) & np.uint32(1)).astype(np.int32)
    tab = np.where(bit == 0, _MAX_EXPERT, _TARGET_VALUE).astype(np.int32)
    return np.ascontiguousarray(tab.reshape(n_tokens, k).T)


def _gate_block(x_ref, w_ref, rep_ref, idx_ref, wgt_ref):
    f32, bf16, i32 = jnp.float32, jnp.bfloat16, jnp.int32
    x = x_ref[...]
    w = w_ref[...]
    logits = jax.lax.dot_general(
        w, x, (((1,), (1,)), ((), ())),
        preferred_element_type=f32).astype(bf16)
    m = jnp.max(logits, axis=0, keepdims=True)
    l32 = logits.astype(f32)
    m32 = jnp.broadcast_to(m, logits.shape).astype(f32)
    e32 = jnp.exp(l32 - m32)
    den = jnp.sum(e32, axis=0, keepdims=True).astype(bf16)
    s32 = e32 / jnp.broadcast_to(den, e32.shape).astype(f32)

    bits = jax.lax.bitcast_convert_type(s32, i32)
    bits = (bits & jnp.int32(0x7FFFFFFF)) ^ (bits >> 31)
    iota = jax.lax.broadcasted_iota(i32, s32.shape, 0)
    key = (bits | jnp.int32(0xFFFF)) ^ iota

    neg = jnp.int32(-2147483648)
    keys8 = []
    for _ in range(_TOPK):
        kmax = jnp.max(key, axis=0, keepdims=True)
        keys8.append(kmax)
        key = jnp.where(key == jnp.broadcast_to(kmax, key.shape), neg, key)
    k8 = jnp.concatenate(keys8, axis=0)
    ti = (k8 ^ jnp.int32(0xFFFF)) & jnp.int32(0xFFFF)
    wbits = ((k8 & jnp.int32(0x7FFFFFFF)) ^ (k8 >> 31)) & jnp.int32(-65536)
    tw32 = jax.lax.bitcast_convert_type(wbits, f32)
    denom = jnp.sum(tw32, axis=0, keepdims=True).astype(bf16)
    tv = (tw32 / jnp.broadcast_to(denom, tw32.shape).astype(f32)).astype(bf16)
    ti = jnp.where(ti == _MAX_EXPERT, rep_ref[...], ti)
    idx_ref[...] = ti
    wgt_ref[...] = tv


def kernel(hidden_states, weight):
    bsz, seq, h = hidden_states.shape
    n = bsz * seq
    x = hidden_states.reshape(n, h)
    rep = jnp.asarray(_replacement_table_t(n, _TOPK))

    bm = 1024
    grid = (n // bm,)
    ti, tv = pl.pallas_call(
        _gate_block,
        grid=grid,
        in_specs=[
            pl.BlockSpec((bm, h), lambda i: (i, 0)),
            pl.BlockSpec((_E, h), lambda i: (0, 0)),
            pl.BlockSpec((_TOPK, bm), lambda i: (0, i)),
        ],
        out_specs=[
            pl.BlockSpec((_TOPK, bm), lambda i: (0, i)),
            pl.BlockSpec((_TOPK, bm), lambda i: (0, i)),
        ],
        out_shape=[
            jax.ShapeDtypeStruct((_TOPK, n), jnp.int32),
            jax.ShapeDtypeStruct((_TOPK, n), hidden_states.dtype),
        ],
        compiler_params=pltpu.CompilerParams(
            dimension_semantics=("parallel",),
        ),
    )(x, weight, rep)
    return ti.T, tv.T
